# baseline (speedup 1.0000x reference)
import functools

import jax
import jax.numpy as jnp
from jax import lax
from jax.experimental import pallas as pl
from jax.experimental.pallas import tpu as pltpu
from jax.experimental.pallas import tpu_sc as plsc

_RADIUS2 = 0.2 * 0.2
_K = 32
_B = 4
_N = 8192
_S = 2048
_C = 128
_SB = 256

_NC = 2
_NS = 16
_NW = _NC * _NS
_Q = _B * _S
_QW = _Q // _NW
_L = 16


def _ball_query_kernel(lhs_ref, rhs_ref, idx_ref):
    q = lhs_ref[...]
    p = rhs_ref[...]
    qb = q[:, 0:3].astype(jnp.bfloat16)
    pb = p[0:3, :].astype(jnp.bfloat16)
    dot = lax.dot_general(qb, pb, (((1,), (0,)), ((), ())),
                          preferred_element_type=jnp.float32)
    d2 = (q[:, 3:4] + p[3:4, :]) - 2.0 * dot
    niota = lax.broadcasted_iota(jnp.int32, (_SB, _N), 1)
    sent = jnp.int32(_N)
    t = jnp.where(d2 < _RADIUS2, niota, sent)

    m = jnp.min(t, axis=1, keepdims=True)
    first = jnp.where(m == sent, jnp.int32(0), m)
    cols = [first]
    for _ in range(1, _K):
        c = jnp.int32(-(2 ** 31)) - (m + 1)
        r = jnp.min(t + c, axis=1, keepdims=True)
        nm = r - c
        m = jnp.where(m == sent, sent, nm)
        cols.append(jnp.where(m == sent, first, m))
    idx_ref[...] = jnp.concatenate(cols, axis=1)


def _ball_query(lhs_b, rhs_b):
    return pl.pallas_call(
        _ball_query_kernel,
        grid=(_S // _SB,),
        in_specs=[
            pl.BlockSpec((_SB, 4), lambda i: (i, 0)),
            pl.BlockSpec((4, _N), lambda i: (0, 0)),
        ],
        out_specs=pl.BlockSpec((_SB, _K), lambda i: (i, 0)),
        out_shape=jax.ShapeDtypeStruct((_S, _K), jnp.int32),
    )(lhs_b, rhs_b)


_NBUF = 8
_QWB = _S // _NW


def _sc_gather_max_kernel(featsT_hbm, idx_hbm, out_hbm, idx_v, rows_v, pool_v,
                          *sems):
    wid = lax.axis_index("s") * _NC + lax.axis_index("c")
    base = wid * _QWB
    pltpu.sync_copy(idx_hbm.at[pl.ds(base, _QWB)], idx_v)

    for j in range(_NBUF):
        pltpu.async_copy(featsT_hbm.at[idx_v.at[j]], rows_v.at[j], sems[j])

    def body(g, carry):
        for j in range(_NBUF):
            q = g * _NBUF + j
            pltpu.make_async_copy(
                featsT_hbm.at[idx_v.at[0]], rows_v.at[j], sems[j]).wait()
            for c in range(_C // _L):
                sl = pl.ds(c * _L, _L)
                acc = rows_v[j, 0, sl]
                for k in range(1, _K):
                    acc = jnp.maximum(acc, rows_v[j, k, sl])
                pool_v[q, sl] = acc
            nq = jnp.minimum(q + _NBUF, _QWB - 1)
            pltpu.async_copy(featsT_hbm.at[idx_v.at[nq]], rows_v.at[j], sems[j])
        return carry

    lax.fori_loop(0, _QWB // _NBUF, body, 0)
    for j in range(_NBUF):
        pltpu.make_async_copy(
            featsT_hbm.at[idx_v.at[0]], rows_v.at[j], sems[j]).wait()
    pltpu.sync_copy(pool_v, out_hbm.at[pl.ds(base, _QWB)])


def _sc_gather_max(featsT_b, idx_b):
    mesh = plsc.VectorSubcoreMesh(core_axis_name="c", subcore_axis_name="s")
    kern = functools.partial(
        pl.kernel,
        mesh=mesh,
        out_type=jax.ShapeDtypeStruct((_S, _C), jnp.float32),
        scratch_types=[
            pltpu.VMEM((_QWB, _K), jnp.int32),
            pltpu.VMEM((_NBUF, _K, _C), jnp.float32),
            pltpu.VMEM((_QWB, _C), jnp.float32),
        ] + [pltpu.SemaphoreType.DMA] * _NBUF,
    )(_sc_gather_max_kernel)
    return kern(featsT_b, idx_b)


def _head_kernel(pool_ref, wt_ref, gamma_ref, beta_ref, out_ref):
    y = jnp.dot(pool_ref[...], wt_ref[...], preferred_element_type=jnp.float32)
    mean = jnp.mean(y, axis=0, keepdims=True)
    var = jnp.mean((y - mean) * (y - mean), axis=0, keepdims=True)
    yn = (y - mean) / jnp.sqrt(var + 1e-5)
    yn = yn * gamma_ref[...] + beta_ref[...]
    out_ref[...] = jnp.where(yn > 0, yn, 0.2 * yn)


def _head(pooled, wt, gamma, beta):
    return pl.pallas_call(
        _head_kernel,
        out_shape=jax.ShapeDtypeStruct((_Q, _C), jnp.float32),
    )(pooled, wt, gamma, beta)


def kernel(xyz, feats, new_xyz, W, gamma, beta):
    sq_x = jnp.sum(xyz * xyz, axis=-1)
    sq_n = jnp.sum(new_xyz * new_xyz, axis=-1)
    lhs = jnp.concatenate([new_xyz, sq_n[..., None]], axis=-1)
    rhs_rows = jnp.concatenate([xyz, sq_x[..., None]], axis=-1)
    rhs = jnp.transpose(rhs_rows, (0, 2, 1))

    featsT = jnp.transpose(feats, (0, 2, 1))
    pooled = jnp.concatenate(
        [_sc_gather_max(featsT[b], _ball_query(lhs[b], rhs[b]))
         for b in range(_B)], axis=0)

    y = _head(pooled, W.T, gamma.reshape(1, _C), beta.reshape(1, _C))
    return jnp.transpose(y.reshape(_B, _S, _C), (0, 2, 1))


# speedup vs baseline: 1.0006x; 1.0006x over previous
import functools

import jax
import jax.numpy as jnp
from jax import lax
from jax.experimental import pallas as pl
from jax.experimental.pallas import tpu as pltpu
from jax.experimental.pallas import tpu_sc as plsc

_RADIUS2 = 0.2 * 0.2
_K = 32
_B = 4
_N = 8192
_S = 2048
_C = 128
_SB = 256

_NC = 2
_NS = 16
_NW = _NC * _NS
_Q = _B * _S
_QW = _Q // _NW
_L = 16


def _ball_query_kernel(lhs_ref, rhs_ref, idx_ref):
    q = lhs_ref[...]
    p = rhs_ref[...]
    qb = q[:, 0:3].astype(jnp.bfloat16)
    pb = p[0:3, :].astype(jnp.bfloat16)
    dot = lax.dot_general(qb, pb, (((1,), (0,)), ((), ())),
                          preferred_element_type=jnp.float32)
    d2 = (q[:, 3:4] + p[3:4, :]) - 2.0 * dot
    niota = lax.broadcasted_iota(jnp.int32, (_SB, _N), 1)
    sent = jnp.int32(_N)
    t = jnp.where(d2 < _RADIUS2, niota, sent)

    m = jnp.min(t, axis=1, keepdims=True)
    first = jnp.where(m == sent, jnp.int32(0), m)
    cols = [first]
    for _ in range(1, _K):
        c = jnp.int32(-(2 ** 31)) - (m + 1)
        r = jnp.min(t + c, axis=1, keepdims=True)
        nm = r - c
        m = jnp.where(m == sent, sent, nm)
        cols.append(jnp.where(m == sent, first, m))
    idx_ref[...] = jnp.concatenate(cols, axis=1)


def _ball_query(lhs_b, rhs_b):
    return pl.pallas_call(
        _ball_query_kernel,
        grid=(_S // _SB,),
        in_specs=[
            pl.BlockSpec((_SB, 4), lambda i: (i, 0)),
            pl.BlockSpec((4, _N), lambda i: (0, 0)),
        ],
        out_specs=pl.BlockSpec((_SB, _K), lambda i: (i, 0)),
        out_shape=jax.ShapeDtypeStruct((_S, _K), jnp.int32),
        compiler_params=pltpu.CompilerParams(
            dimension_semantics=("parallel",)),
    )(lhs_b, rhs_b)


_NBUF = 8
_QWB = _S // _NW


def _sc_gather_max_kernel(featsT_hbm, idx_hbm, out_hbm, idx_v, rows_v, pool_v,
                          *sems):
    wid = lax.axis_index("s") * _NC + lax.axis_index("c")
    base = wid * _QWB
    pltpu.sync_copy(idx_hbm.at[pl.ds(base, _QWB)], idx_v)

    for j in range(_NBUF):
        pltpu.async_copy(featsT_hbm.at[idx_v.at[j]], rows_v.at[j], sems[j])

    def body(g, carry):
        for j in range(_NBUF):
            q = g * _NBUF + j
            pltpu.make_async_copy(
                featsT_hbm.at[idx_v.at[0]], rows_v.at[j], sems[j]).wait()
            for c in range(_C // _L):
                sl = pl.ds(c * _L, _L)
                acc = rows_v[j, 0, sl]
                for k in range(1, _K):
                    acc = jnp.maximum(acc, rows_v[j, k, sl])
                pool_v[q, sl] = acc
            nq = jnp.minimum(q + _NBUF, _QWB - 1)
            pltpu.async_copy(featsT_hbm.at[idx_v.at[nq]], rows_v.at[j], sems[j])
        return carry

    lax.fori_loop(0, _QWB // _NBUF, body, 0)
    for j in range(_NBUF):
        pltpu.make_async_copy(
            featsT_hbm.at[idx_v.at[0]], rows_v.at[j], sems[j]).wait()
    pltpu.sync_copy(pool_v, out_hbm.at[pl.ds(base, _QWB)])


def _sc_gather_max(featsT_b, idx_b):
    mesh = plsc.VectorSubcoreMesh(core_axis_name="c", subcore_axis_name="s")
    kern = functools.partial(
        pl.kernel,
        mesh=mesh,
        out_type=jax.ShapeDtypeStruct((_S, _C), jnp.float32),
        scratch_types=[
            pltpu.VMEM((_QWB, _K), jnp.int32),
            pltpu.VMEM((_NBUF, _K, _C), jnp.float32),
            pltpu.VMEM((_QWB, _C), jnp.float32),
        ] + [pltpu.SemaphoreType.DMA] * _NBUF,
    )(_sc_gather_max_kernel)
    return kern(featsT_b, idx_b)


def _head_kernel(pool_ref, wt_ref, gamma_ref, beta_ref, out_ref):
    y = jnp.dot(pool_ref[...], wt_ref[...], preferred_element_type=jnp.float32)
    mean = jnp.mean(y, axis=0, keepdims=True)
    var = jnp.mean((y - mean) * (y - mean), axis=0, keepdims=True)
    yn = (y - mean) / jnp.sqrt(var + 1e-5)
    yn = yn * gamma_ref[...] + beta_ref[...]
    out_ref[...] = jnp.where(yn > 0, yn, 0.2 * yn)


def _head(pooled, wt, gamma, beta):
    return pl.pallas_call(
        _head_kernel,
        out_shape=jax.ShapeDtypeStruct((_Q, _C), jnp.float32),
    )(pooled, wt, gamma, beta)


def kernel(xyz, feats, new_xyz, W, gamma, beta):
    sq_x = jnp.sum(xyz * xyz, axis=-1)
    sq_n = jnp.sum(new_xyz * new_xyz, axis=-1)
    lhs = jnp.concatenate([new_xyz, sq_n[..., None]], axis=-1)
    rhs_rows = jnp.concatenate([xyz, sq_x[..., None]], axis=-1)
    rhs = jnp.transpose(rhs_rows, (0, 2, 1))

    featsT = jnp.transpose(feats, (0, 2, 1))
    pooled = jnp.concatenate(
        [_sc_gather_max(featsT[b], _ball_query(lhs[b], rhs[b]))
         for b in range(_B)], axis=0)

    y = _head(pooled, W.T, gamma.reshape(1, _C), beta.reshape(1, _C))
    return jnp.transpose(y.reshape(_B, _S, _C), (0, 2, 1))
